# Initial kernel scaffold; baseline (speedup 1.0000x reference)
import jax
import jax.numpy as jnp
from jax.experimental import pallas as pl


def kernel(inputs, embed_table, W_weight, W_bias):
    raise NotImplementedError("write your pallas kernel here")


# speedup vs baseline: 2.9342x; 2.9342x over previous
import functools

import jax
import jax.numpy as jnp
from jax import lax
from jax.experimental import pallas as pl
from jax.experimental.pallas import tpu as pltpu
from jax.experimental.pallas import tpu_sc as plsc

B = 16384
CTX = 200
D = 64
NCLS = 1000

NC, NS = 2, 16
NW = NC * NS
SPW = B // NW
IDX_BLK = 128
NIB = SPW // IDX_BLK
NBUF = 4
G1 = 128
G2 = CTX - G1

_mesh = plsc.VectorSubcoreMesh(
    core_axis_name="c", subcore_axis_name="s", num_cores=NC, num_subcores=NS
)


@functools.partial(
    pl.kernel,
    out_type=jax.ShapeDtypeStruct((B, D), jnp.float32),
    mesh=_mesh,
    scratch_types=[
        pltpu.VMEM((IDX_BLK, CTX), jnp.int32),
        pltpu.VMEM((NBUF, CTX, D), jnp.float32),
        pltpu.VMEM((IDX_BLK, D), jnp.float32),
    ]
    + [pltpu.SemaphoreType.DMA] * NBUF,
    compiler_params=pltpu.CompilerParams(use_tc_tiling_on_sc=False),
)
def _pool(inputs_hbm, table_hbm, out_hbm, idx_v, rows_v, out_v, *sems):
    wid = lax.axis_index("s") * NC + lax.axis_index("c")
    base = wid * SPW

    def fire(s, b):
        pltpu.async_copy(
            table_hbm.at[idx_v.at[s, pl.ds(0, G1)]],
            rows_v.at[b, pl.ds(0, G1)],
            sems[b],
        )
        pltpu.async_copy(
            table_hbm.at[idx_v.at[s, pl.ds(G1, G2)]],
            rows_v.at[b, pl.ds(G1, G2)],
            sems[b],
        )

    def drain(s, b):
        pltpu.make_async_copy(
            table_hbm.at[idx_v.at[s, pl.ds(0, G1)]],
            rows_v.at[b, pl.ds(0, G1)],
            sems[b],
        ).wait()
        pltpu.make_async_copy(
            table_hbm.at[idx_v.at[s, pl.ds(G1, G2)]],
            rows_v.at[b, pl.ds(G1, G2)],
            sems[b],
        ).wait()

    for ib in range(NIB):
        blk0 = base + ib * IDX_BLK
        pltpu.sync_copy(inputs_hbm.at[pl.ds(blk0, IDX_BLK)], idx_v)
        for b in range(NBUF):
            fire(b, b)

        def blk_body(i, _):
            s0 = i * NBUF
            for b in range(NBUF):
                s = s0 + b
                drain(s, b)

                def acc_body(r, carry, b=b):
                    a0, a1, a2, a3 = carry
                    a0 = a0 + rows_v[b, r, pl.ds(0, 16)]
                    a1 = a1 + rows_v[b, r, pl.ds(16, 16)]
                    a2 = a2 + rows_v[b, r, pl.ds(32, 16)]
                    a3 = a3 + rows_v[b, r, pl.ds(48, 16)]
                    return (a0, a1, a2, a3)

                z = jnp.zeros((16,), jnp.float32)
                a0, a1, a2, a3 = lax.fori_loop(0, CTX, acc_body, (z, z, z, z))
                out_v[s, pl.ds(0, 16)] = a0
                out_v[s, pl.ds(16, 16)] = a1
                out_v[s, pl.ds(32, 16)] = a2
                out_v[s, pl.ds(48, 16)] = a3

                @pl.when(s + NBUF < IDX_BLK)
                def _(s=s, b=b):
                    fire(s + NBUF, b)

            return 0

        lax.fori_loop(0, IDX_BLK // NBUF, blk_body, 0)
        pltpu.sync_copy(out_v, out_hbm.at[pl.ds(blk0, IDX_BLK)])


BM = 1024


def _mm_body(x_ref, w_ref, b_ref, o_ref):
    o_ref[...] = (
        jnp.dot(x_ref[...], w_ref[...], preferred_element_type=jnp.float32)
        + b_ref[...]
    )


def _matmul(pooled, wt, bias2d):
    return pl.pallas_call(
        _mm_body,
        grid=(B // BM,),
        in_specs=[
            pl.BlockSpec((BM, D), lambda i: (i, 0)),
            pl.BlockSpec((D, NCLS), lambda i: (0, 0)),
            pl.BlockSpec((1, NCLS), lambda i: (0, 0)),
        ],
        out_specs=pl.BlockSpec((BM, NCLS), lambda i: (i, 0)),
        out_shape=jax.ShapeDtypeStruct((B, NCLS), jnp.float32),
    )(pooled, wt, bias2d)


def kernel(inputs, embed_table, W_weight, W_bias):
    idx = inputs.astype(jnp.int32)
    pooled = _pool(idx, embed_table)
    return _matmul(pooled, W_weight.T, W_bias[None, :])
